# initial kernel scaffold (unmeasured)
import jax
import jax.numpy as jnp
from jax import lax
from jax.experimental import pallas as pl
from jax.experimental.pallas import tpu as pltpu


def kernel(
    x,
):
    def body(*refs):
        pass

    out_shape = jax.ShapeDtypeStruct(..., jnp.float32)
    return pl.pallas_call(body, out_shape=out_shape)(...)


# baseline (device time: 72137 ns/iter reference)
import jax
import jax.numpy as jnp
from jax import lax
from jax.experimental import pallas as pl
from jax.experimental.pallas import tpu as pltpu

N_DEV = 4
BLOCK_ROWS = 256


def _totals_body(x_ref, t_ref):
    b = pl.program_id(0)
    blk = x_ref[...]
    r = blk.shape[0]
    while r > 1:
        h = r // 2
        blk = blk[:h, :] * blk[h:r, :]
        r = h

    @pl.when(b == 0)
    def _():
        t_ref[...] = blk

    @pl.when(b > 0)
    def _():
        t_ref[...] = t_ref[...] * blk


def _collective_body(t_ref, p_ref, comm_ref, send_sems, recv_sems):
    my = lax.axis_index("i")
    left = lax.rem(my + N_DEV - 1, N_DEV)
    right = lax.rem(my + 1, N_DEV)
    n = t_ref.shape[1]

    barrier_sem = pltpu.get_barrier_semaphore()
    for nbr in (left, right):
        pl.semaphore_signal(
            barrier_sem, inc=1,
            device_id=(nbr,), device_id_type=pl.DeviceIdType.MESH,
        )
    pl.semaphore_wait(barrier_sem, 2)

    comm_ref[0] = jnp.broadcast_to(t_ref[...], (8, n))

    p = jnp.ones((1, n), jnp.float32)
    for h in range(N_DEV - 1):
        rdma = pltpu.make_async_remote_copy(
            src_ref=comm_ref.at[h],
            dst_ref=comm_ref.at[h + 1],
            send_sem=send_sems.at[h],
            recv_sem=recv_sems.at[h],
            device_id=(right,),
            device_id_type=pl.DeviceIdType.MESH,
        )
        rdma.start()
        rdma.wait()
        chunk = comm_ref[h + 1, 0:1, :]
        p = p * jnp.where(my > h, chunk, jnp.ones_like(chunk))
    p_ref[...] = p


def _scan_body(x_ref, p_ref, out_ref, carry_ref):
    b = pl.program_id(0)

    @pl.when(b == 0)
    def _():
        carry_ref[...] = p_ref[...]

    acc = x_ref[...]
    rows, n = acc.shape
    d = 1
    while d < rows:
        shifted = jnp.concatenate(
            [jnp.ones((d, n), jnp.float32), acc[: rows - d, :]], axis=0
        )
        acc = acc * shifted
        d *= 2
    acc = acc * carry_ref[...]
    out_ref[...] = acc
    carry_ref[...] = acc[rows - 1 : rows, :]


def kernel(x):
    m, n = x.shape
    nb = m // BLOCK_ROWS

    totals = pl.pallas_call(
        _totals_body,
        grid=(nb,),
        in_specs=[
            pl.BlockSpec((BLOCK_ROWS, n), lambda b: (b, 0),
                         memory_space=pltpu.VMEM),
        ],
        out_specs=pl.BlockSpec((1, n), lambda b: (0, 0),
                               memory_space=pltpu.VMEM),
        out_shape=jax.ShapeDtypeStruct((1, n), jnp.float32),
        compiler_params=pltpu.CompilerParams(
            dimension_semantics=("arbitrary",),
        ),
    )(x)

    prefix = pl.pallas_call(
        _collective_body,
        in_specs=[pl.BlockSpec(memory_space=pltpu.VMEM)],
        out_specs=pl.BlockSpec(memory_space=pltpu.VMEM),
        out_shape=jax.ShapeDtypeStruct((1, n), jnp.float32),
        scratch_shapes=[
            pltpu.VMEM((N_DEV, 8, n), jnp.float32),
            pltpu.SemaphoreType.DMA((N_DEV - 1,)),
            pltpu.SemaphoreType.DMA((N_DEV - 1,)),
        ],
        compiler_params=pltpu.CompilerParams(collective_id=0),
    )(totals)

    return pl.pallas_call(
        _scan_body,
        grid=(nb,),
        in_specs=[
            pl.BlockSpec((BLOCK_ROWS, n), lambda b: (b, 0),
                         memory_space=pltpu.VMEM),
            pl.BlockSpec((1, n), lambda b: (0, 0),
                         memory_space=pltpu.VMEM),
        ],
        out_specs=pl.BlockSpec((BLOCK_ROWS, n), lambda b: (b, 0),
                               memory_space=pltpu.VMEM),
        out_shape=jax.ShapeDtypeStruct((m, n), jnp.float32),
        scratch_shapes=[pltpu.VMEM((1, n), jnp.float32)],
        compiler_params=pltpu.CompilerParams(
            dimension_semantics=("arbitrary",),
        ),
    )(x, prefix)
